# baseline (device time: 53859 ns/iter reference)
import jax
import jax.numpy as jnp
from jax import lax
from jax.experimental import pallas as pl
from jax.experimental.pallas import tpu as pltpu

N_DEV = 16


def kernel(x, router_W, route_idx, expert_W):
    del router_W
    n_tok, d_in = x.shape
    e_per, _, d_out = expert_W.shape

    def body(x_ref, ridx_ref, ew_ref, out_ref, comm_ref, send_sems, recv_sems):
        my = lax.axis_index("i")
        left = lax.rem(my + N_DEV - 1, N_DEV)
        right = lax.rem(my + 1, N_DEV)

        barrier_sem = pltpu.get_barrier_semaphore()
        for nbr in (left, right):
            pl.semaphore_signal(
                barrier_sem, inc=1,
                device_id=(nbr,), device_id_type=pl.DeviceIdType.MESH,
            )
        pl.semaphore_wait(barrier_sem, 2)

        ridx = ridx_ref[:, :]
        xb = x_ref[:, :].astype(jnp.bfloat16)
        acc = jnp.zeros((n_tok, d_out), jnp.float32)
        for e in range(e_per):
            gid = my * e_per + e
            mask = (ridx == gid).astype(jnp.bfloat16)
            acc = acc + jnp.dot(
                xb * mask,
                ew_ref[e].astype(jnp.bfloat16),
                preferred_element_type=jnp.float32,
            )
        out_ref[:, :] = acc
        comm_ref[0] = acc.astype(jnp.bfloat16)

        for h in range(N_DEV - 1):
            rdma = pltpu.make_async_remote_copy(
                src_ref=comm_ref.at[h],
                dst_ref=comm_ref.at[h + 1],
                send_sem=send_sems.at[h],
                recv_sem=recv_sems.at[h + 1],
                device_id=(right,),
                device_id_type=pl.DeviceIdType.MESH,
            )
            rdma.start()
            rdma.wait()
            out_ref[:, :] += comm_ref[h + 1].astype(jnp.float32)

    return pl.pallas_call(
        body,
        out_shape=jax.ShapeDtypeStruct((n_tok, d_out), jnp.float32),
        in_specs=[
            pl.BlockSpec(memory_space=pltpu.VMEM),
            pl.BlockSpec(memory_space=pltpu.VMEM),
            pl.BlockSpec(memory_space=pltpu.VMEM),
        ],
        out_specs=pl.BlockSpec(memory_space=pltpu.VMEM),
        scratch_shapes=[
            pltpu.VMEM((N_DEV, n_tok, d_out), jnp.bfloat16),
            pltpu.SemaphoreType.DMA((N_DEV,)),
            pltpu.SemaphoreType.DMA((N_DEV,)),
        ],
        compiler_params=pltpu.CompilerParams(collective_id=0),
    )(x, route_idx, expert_W)


# device time: 22116 ns/iter; 2.4353x vs baseline; 2.4353x over previous
import jax
import jax.numpy as jnp
from jax import lax
from jax.experimental import pallas as pl
from jax.experimental.pallas import tpu as pltpu

N_DEV = 16
N_ROUNDS = 4


def kernel(x, router_W, route_idx, expert_W):
    del router_W
    n_tok, d_in = x.shape
    e_per, _, d_out = expert_W.shape

    def body(x_ref, ridx_ref, ew_ref, out_ref,
             send_ref, recv_ref, send_sems, recv_sems):
        my = lax.axis_index("i")

        barrier_sem = pltpu.get_barrier_semaphore()
        for r in range(N_ROUNDS):
            pl.semaphore_signal(
                barrier_sem, inc=1,
                device_id=(my ^ (1 << r),),
                device_id_type=pl.DeviceIdType.MESH,
            )
        pl.semaphore_wait(barrier_sem, N_ROUNDS)

        ridx = ridx_ref[:, :]
        xb = x_ref[:, :].astype(jnp.bfloat16)
        acc = jnp.zeros((n_tok, d_out), jnp.float32)
        for e in range(e_per):
            gid = my * e_per + e
            mask = (ridx == gid).astype(jnp.bfloat16)
            acc = acc + jnp.dot(
                xb * mask,
                ew_ref[e].astype(jnp.bfloat16),
                preferred_element_type=jnp.float32,
            )

        for r in range(N_ROUNDS):
            partner = my ^ (1 << r)
            send_ref[r] = acc.astype(jnp.bfloat16)
            rdma = pltpu.make_async_remote_copy(
                src_ref=send_ref.at[r],
                dst_ref=recv_ref.at[r],
                send_sem=send_sems.at[r],
                recv_sem=recv_sems.at[r],
                device_id=(partner,),
                device_id_type=pl.DeviceIdType.MESH,
            )
            rdma.start()
            rdma.wait()
            acc = acc + recv_ref[r].astype(jnp.float32)

        out_ref[:, :] = acc

    return pl.pallas_call(
        body,
        out_shape=jax.ShapeDtypeStruct((n_tok, d_out), jnp.float32),
        in_specs=[
            pl.BlockSpec(memory_space=pltpu.VMEM),
            pl.BlockSpec(memory_space=pltpu.VMEM),
            pl.BlockSpec(memory_space=pltpu.VMEM),
        ],
        out_specs=pl.BlockSpec(memory_space=pltpu.VMEM),
        scratch_shapes=[
            pltpu.VMEM((N_ROUNDS, n_tok, d_out), jnp.bfloat16),
            pltpu.VMEM((N_ROUNDS, n_tok, d_out), jnp.bfloat16),
            pltpu.SemaphoreType.DMA((N_ROUNDS,)),
            pltpu.SemaphoreType.DMA((N_ROUNDS,)),
        ],
        compiler_params=pltpu.CompilerParams(collective_id=0),
    )(x, route_idx, expert_W)


# device time: 18922 ns/iter; 2.8464x vs baseline; 1.1688x over previous
import jax
import jax.numpy as jnp
from jax import lax
from jax.experimental import pallas as pl
from jax.experimental.pallas import tpu as pltpu

N_DEV = 16
PLANE = 4
NZ = 4


def kernel(x, router_W, route_idx, expert_W):
    del router_W
    n_tok, d_in = x.shape
    e_per, _, d_out = expert_W.shape

    def body(x_ref, ridx_ref, ew_ref, out_ref,
             commA, commB, send_semsA, recv_semsA, send_semsB, recv_semsB):
        my = lax.axis_index("i")
        my_p = lax.rem(my, PLANE)
        my_z = lax.div(my, PLANE)

        barrier_sem = pltpu.get_barrier_semaphore()
        for dp in range(1, PLANE):
            pl.semaphore_signal(
                barrier_sem, inc=1,
                device_id=(my_z * PLANE + lax.rem(my_p + dp, PLANE),),
                device_id_type=pl.DeviceIdType.MESH,
            )
        for dz in range(1, NZ):
            pl.semaphore_signal(
                barrier_sem, inc=1,
                device_id=(lax.rem(my_z + dz, NZ) * PLANE + my_p,),
                device_id_type=pl.DeviceIdType.MESH,
            )
        pl.semaphore_wait(barrier_sem, PLANE - 1 + NZ - 1)

        ridx = ridx_ref[:, :]
        xb = x_ref[:, :].astype(jnp.bfloat16)
        acc = jnp.zeros((n_tok, d_out), jnp.float32)
        for e in range(e_per):
            gid = my * e_per + e
            mask = (ridx == gid).astype(jnp.bfloat16)
            acc = acc + jnp.dot(
                xb * mask,
                ew_ref[e].astype(jnp.bfloat16),
                preferred_element_type=jnp.float32,
            )

        commA[my_p] = acc.astype(jnp.bfloat16)
        rdmasA = []
        for dp in range(1, PLANE):
            tgt = my_z * PLANE + lax.rem(my_p + dp, PLANE)
            rdma = pltpu.make_async_remote_copy(
                src_ref=commA.at[my_p],
                dst_ref=commA.at[my_p],
                send_sem=send_semsA.at[dp],
                recv_sem=recv_semsA.at[my_p],
                device_id=(tgt,),
                device_id_type=pl.DeviceIdType.MESH,
            )
            rdma.start()
            rdmasA.append(rdma)
        for dp in range(1, PLANE):
            src_p = lax.rem(my_p + dp, PLANE)
            recv = pltpu.make_async_remote_copy(
                src_ref=commA.at[my_p],
                dst_ref=commA.at[src_p],
                send_sem=send_semsA.at[dp],
                recv_sem=recv_semsA.at[src_p],
                device_id=(my,),
                device_id_type=pl.DeviceIdType.MESH,
            )
            recv.wait_recv()
        acc = (commA[0].astype(jnp.float32) + commA[1].astype(jnp.float32)
               + commA[2].astype(jnp.float32) + commA[3].astype(jnp.float32))

        commB[my_z] = acc.astype(jnp.bfloat16)
        rdmasB = []
        for dz in range(1, NZ):
            tgt = lax.rem(my_z + dz, NZ) * PLANE + my_p
            rdma = pltpu.make_async_remote_copy(
                src_ref=commB.at[my_z],
                dst_ref=commB.at[my_z],
                send_sem=send_semsB.at[dz],
                recv_sem=recv_semsB.at[my_z],
                device_id=(tgt,),
                device_id_type=pl.DeviceIdType.MESH,
            )
            rdma.start()
            rdmasB.append(rdma)
        for dz in range(1, NZ):
            src_z = lax.rem(my_z + dz, NZ)
            recv = pltpu.make_async_remote_copy(
                src_ref=commB.at[my_z],
                dst_ref=commB.at[src_z],
                send_sem=send_semsB.at[dz],
                recv_sem=recv_semsB.at[src_z],
                device_id=(my,),
                device_id_type=pl.DeviceIdType.MESH,
            )
            recv.wait_recv()
        out_ref[:, :] = (
            commB[0].astype(jnp.float32) + commB[1].astype(jnp.float32)
            + commB[2].astype(jnp.float32) + commB[3].astype(jnp.float32)
        )

        for rdma in rdmasA + rdmasB:
            rdma.wait_send()

    return pl.pallas_call(
        body,
        out_shape=jax.ShapeDtypeStruct((n_tok, d_out), jnp.float32),
        in_specs=[
            pl.BlockSpec(memory_space=pltpu.VMEM),
            pl.BlockSpec(memory_space=pltpu.VMEM),
            pl.BlockSpec(memory_space=pltpu.VMEM),
        ],
        out_specs=pl.BlockSpec(memory_space=pltpu.VMEM),
        scratch_shapes=[
            pltpu.VMEM((PLANE, n_tok, d_out), jnp.bfloat16),
            pltpu.VMEM((NZ, n_tok, d_out), jnp.bfloat16),
            pltpu.SemaphoreType.DMA((PLANE,)),
            pltpu.SemaphoreType.DMA((PLANE,)),
            pltpu.SemaphoreType.DMA((NZ,)),
            pltpu.SemaphoreType.DMA((NZ,)),
        ],
        compiler_params=pltpu.CompilerParams(collective_id=0),
    )(x, route_idx, expert_W)


# device time: 17421 ns/iter; 3.0916x vs baseline; 1.0862x over previous
import jax
import jax.numpy as jnp
from jax import lax
from jax.experimental import pallas as pl
from jax.experimental.pallas import tpu as pltpu

N_DEV = 16
PLANE = 4
NZ = 4


def kernel(x, router_W, route_idx, expert_W):
    del router_W
    n_tok, d_in = x.shape
    e_per, _, d_out = expert_W.shape
    blk = n_tok // PLANE

    def body(x_ref, ridx_ref, ew_ref, out_ref,
             part_ref, commA, commB,
             sendA, recvA, sendB, recvB, sendC, recvC):
        my = lax.axis_index("i")
        my_p = lax.rem(my, PLANE)
        my_z = lax.div(my, PLANE)

        barrier_sem = pltpu.get_barrier_semaphore()
        for dp in range(1, PLANE):
            pl.semaphore_signal(
                barrier_sem, inc=1,
                device_id=(my_z * PLANE + lax.rem(my_p + dp, PLANE),),
                device_id_type=pl.DeviceIdType.MESH,
            )
        for dz in range(1, NZ):
            pl.semaphore_signal(
                barrier_sem, inc=1,
                device_id=(lax.rem(my_z + dz, NZ) * PLANE + my_p,),
                device_id_type=pl.DeviceIdType.MESH,
            )
        pl.semaphore_wait(barrier_sem, PLANE - 1 + NZ - 1)

        ridx = ridx_ref[:, :]
        xb = x_ref[:, :].astype(jnp.bfloat16)
        acc = jnp.zeros((n_tok, d_out), jnp.float32)
        for e in range(e_per):
            gid = my * e_per + e
            mask = (ridx == gid).astype(jnp.bfloat16)
            acc = acc + jnp.dot(
                xb * mask,
                ew_ref[e].astype(jnp.bfloat16),
                preferred_element_type=jnp.float32,
            )
        part_ref[:, :] = acc.astype(jnp.bfloat16)

        pending = []

        for dp in range(1, PLANE):
            tgt_p = lax.rem(my_p + dp, PLANE)
            rdma = pltpu.make_async_remote_copy(
                src_ref=part_ref.at[pl.ds(tgt_p * blk, blk), :],
                dst_ref=commA.at[my_p],
                send_sem=sendA.at[dp],
                recv_sem=recvA.at[my_p],
                device_id=(my_z * PLANE + tgt_p,),
                device_id_type=pl.DeviceIdType.MESH,
            )
            rdma.start()
            pending.append(rdma)
        q = part_ref[pl.ds(my_p * blk, blk), :].astype(jnp.float32)
        for dp in range(1, PLANE):
            src_p = lax.rem(my_p + dp, PLANE)
            recv = pltpu.make_async_remote_copy(
                src_ref=part_ref.at[pl.ds(0, blk), :],
                dst_ref=commA.at[src_p],
                send_sem=sendA.at[dp],
                recv_sem=recvA.at[src_p],
                device_id=(my,),
                device_id_type=pl.DeviceIdType.MESH,
            )
            recv.wait_recv()
            q = q + commA[src_p].astype(jnp.float32)

        commB[my_z] = q.astype(jnp.bfloat16)
        for dz in range(1, NZ):
            tgt = lax.rem(my_z + dz, NZ) * PLANE + my_p
            rdma = pltpu.make_async_remote_copy(
                src_ref=commB.at[my_z],
                dst_ref=commB.at[my_z],
                send_sem=sendB.at[dz],
                recv_sem=recvB.at[my_z],
                device_id=(tgt,),
                device_id_type=pl.DeviceIdType.MESH,
            )
            rdma.start()
            pending.append(rdma)
        t_blk = q
        for dz in range(1, NZ):
            src_z = lax.rem(my_z + dz, NZ)
            recv = pltpu.make_async_remote_copy(
                src_ref=commB.at[my_z],
                dst_ref=commB.at[src_z],
                send_sem=sendB.at[dz],
                recv_sem=recvB.at[src_z],
                device_id=(my,),
                device_id_type=pl.DeviceIdType.MESH,
            )
            recv.wait_recv()
            t_blk = t_blk + commB[src_z].astype(jnp.float32)

        out_ref[pl.ds(my_p * blk, blk), :] = t_blk
        for dp in range(1, PLANE):
            tgt_p = lax.rem(my_p + dp, PLANE)
            rdma = pltpu.make_async_remote_copy(
                src_ref=out_ref.at[pl.ds(my_p * blk, blk), :],
                dst_ref=out_ref.at[pl.ds(my_p * blk, blk), :],
                send_sem=sendC.at[dp],
                recv_sem=recvC.at[my_p],
                device_id=(my_z * PLANE + tgt_p,),
                device_id_type=pl.DeviceIdType.MESH,
            )
            rdma.start()
            pending.append(rdma)
        for dp in range(1, PLANE):
            src_p = lax.rem(my_p + dp, PLANE)
            recv = pltpu.make_async_remote_copy(
                src_ref=out_ref.at[pl.ds(my_p * blk, blk), :],
                dst_ref=out_ref.at[pl.ds(src_p * blk, blk), :],
                send_sem=sendC.at[dp],
                recv_sem=recvC.at[src_p],
                device_id=(my,),
                device_id_type=pl.DeviceIdType.MESH,
            )
            recv.wait_recv()

        for rdma in pending:
            rdma.wait_send()

    return pl.pallas_call(
        body,
        out_shape=jax.ShapeDtypeStruct((n_tok, d_out), jnp.float32),
        in_specs=[
            pl.BlockSpec(memory_space=pltpu.VMEM),
            pl.BlockSpec(memory_space=pltpu.VMEM),
            pl.BlockSpec(memory_space=pltpu.VMEM),
        ],
        out_specs=pl.BlockSpec(memory_space=pltpu.VMEM),
        scratch_shapes=[
            pltpu.VMEM((n_tok, d_out), jnp.bfloat16),
            pltpu.VMEM((PLANE, blk, d_out), jnp.bfloat16),
            pltpu.VMEM((NZ, blk, d_out), jnp.bfloat16),
            pltpu.SemaphoreType.DMA((PLANE,)),
            pltpu.SemaphoreType.DMA((PLANE,)),
            pltpu.SemaphoreType.DMA((NZ,)),
            pltpu.SemaphoreType.DMA((NZ,)),
            pltpu.SemaphoreType.DMA((PLANE,)),
            pltpu.SemaphoreType.DMA((PLANE,)),
        ],
        compiler_params=pltpu.CompilerParams(collective_id=0),
    )(x, route_idx, expert_W)


# device time: 15445 ns/iter; 3.4871x vs baseline; 1.1279x over previous
import jax
import jax.numpy as jnp
from jax import lax
from jax.experimental import pallas as pl
from jax.experimental.pallas import tpu as pltpu

N_DEV = 16
PLANE = 4
NZ = 4
DP_ORDER = (1, 3, 2)


def kernel(x, router_W, route_idx, expert_W):
    del router_W
    n_tok, d_in = x.shape
    e_per, _, d_out = expert_W.shape
    blk = n_tok // PLANE

    def body(x_ref, ridx_ref, ew_ref, out_ref,
             part_ref, commA, commB,
             sendA, recvA, sendB, recvB, sendC, recvC, col_sem):
        my = lax.axis_index("i")
        my_p = lax.rem(my, PLANE)
        my_z = lax.div(my, PLANE)

        barrier_sem = pltpu.get_barrier_semaphore()
        for dp in range(1, PLANE):
            pl.semaphore_signal(
                barrier_sem, inc=1,
                device_id=(my_z * PLANE + lax.rem(my_p + dp, PLANE),),
                device_id_type=pl.DeviceIdType.MESH,
            )
        for dz in range(1, NZ):
            pl.semaphore_signal(
                col_sem, inc=1,
                device_id=(lax.rem(my_z + dz, NZ) * PLANE + my_p,),
                device_id_type=pl.DeviceIdType.MESH,
            )

        wcat = jnp.concatenate(
            [ew_ref[0], ew_ref[1]], axis=0
        ).astype(jnp.bfloat16)

        def partial_block(t):
            rows = pl.ds(t * blk, blk)
            xr = x_ref[rows, :].astype(jnp.bfloat16)
            rr = ridx_ref[rows, :]
            m0 = (rr == my * e_per).astype(jnp.bfloat16)
            m1 = (rr == my * e_per + 1).astype(jnp.bfloat16)
            xcat = jnp.concatenate([xr * m0, xr * m1], axis=1)
            return jnp.dot(xcat, wcat, preferred_element_type=jnp.float32)

        pending = []

        pl.semaphore_wait(barrier_sem, PLANE - 1)
        for dp in DP_ORDER:
            tgt_p = lax.rem(my_p + dp, PLANE)
            part_ref[dp - 1] = partial_block(tgt_p).astype(jnp.bfloat16)
            rdma = pltpu.make_async_remote_copy(
                src_ref=part_ref.at[dp - 1],
                dst_ref=commA.at[my_p],
                send_sem=sendA.at[dp],
                recv_sem=recvA.at[my_p],
                device_id=(my_z * PLANE + tgt_p,),
                device_id_type=pl.DeviceIdType.MESH,
            )
            rdma.start()
            pending.append(rdma)
        q = partial_block(my_p)
        for dp in DP_ORDER:
            src_p = lax.rem(my_p + dp, PLANE)
            recv = pltpu.make_async_remote_copy(
                src_ref=part_ref.at[0],
                dst_ref=commA.at[src_p],
                send_sem=sendA.at[dp],
                recv_sem=recvA.at[src_p],
                device_id=(my,),
                device_id_type=pl.DeviceIdType.MESH,
            )
            recv.wait_recv()
            q = q + commA[src_p].astype(jnp.float32)

        pl.semaphore_wait(col_sem, NZ - 1)
        commB[my_z] = q.astype(jnp.bfloat16)
        for dz in range(1, NZ):
            tgt = lax.rem(my_z + dz, NZ) * PLANE + my_p
            rdma = pltpu.make_async_remote_copy(
                src_ref=commB.at[my_z],
                dst_ref=commB.at[my_z],
                send_sem=sendB.at[dz],
                recv_sem=recvB.at[my_z],
                device_id=(tgt,),
                device_id_type=pl.DeviceIdType.MESH,
            )
            rdma.start()
            pending.append(rdma)
        t_blk = q
        for dz in range(1, NZ):
            src_z = lax.rem(my_z + dz, NZ)
            recv = pltpu.make_async_remote_copy(
                src_ref=commB.at[my_z],
                dst_ref=commB.at[src_z],
                send_sem=sendB.at[dz],
                recv_sem=recvB.at[src_z],
                device_id=(my,),
                device_id_type=pl.DeviceIdType.MESH,
            )
            recv.wait_recv()
            t_blk = t_blk + commB[src_z].astype(jnp.float32)

        out_ref[pl.ds(my_p * blk, blk), :] = t_blk
        for dp in DP_ORDER:
            tgt_p = lax.rem(my_p + dp, PLANE)
            rdma = pltpu.make_async_remote_copy(
                src_ref=out_ref.at[pl.ds(my_p * blk, blk), :],
                dst_ref=out_ref.at[pl.ds(my_p * blk, blk), :],
                send_sem=sendC.at[dp],
                recv_sem=recvC.at[my_p],
                device_id=(my_z * PLANE + tgt_p,),
                device_id_type=pl.DeviceIdType.MESH,
            )
            rdma.start()
            pending.append(rdma)
        for dp in DP_ORDER:
            src_p = lax.rem(my_p + dp, PLANE)
            recv = pltpu.make_async_remote_copy(
                src_ref=out_ref.at[pl.ds(my_p * blk, blk), :],
                dst_ref=out_ref.at[pl.ds(src_p * blk, blk), :],
                send_sem=sendC.at[dp],
                recv_sem=recvC.at[src_p],
                device_id=(my,),
                device_id_type=pl.DeviceIdType.MESH,
            )
            recv.wait_recv()

        for rdma in pending:
            rdma.wait_send()

    return pl.pallas_call(
        body,
        out_shape=jax.ShapeDtypeStruct((n_tok, d_out), jnp.float32),
        in_specs=[
            pl.BlockSpec(memory_space=pltpu.VMEM),
            pl.BlockSpec(memory_space=pltpu.VMEM),
            pl.BlockSpec(memory_space=pltpu.VMEM),
        ],
        out_specs=pl.BlockSpec(memory_space=pltpu.VMEM),
        scratch_shapes=[
            pltpu.VMEM((PLANE - 1, blk, d_out), jnp.bfloat16),
            pltpu.VMEM((PLANE, blk, d_out), jnp.bfloat16),
            pltpu.VMEM((NZ, blk, d_out), jnp.bfloat16),
            pltpu.SemaphoreType.DMA((PLANE,)),
            pltpu.SemaphoreType.DMA((PLANE,)),
            pltpu.SemaphoreType.DMA((NZ,)),
            pltpu.SemaphoreType.DMA((NZ,)),
            pltpu.SemaphoreType.DMA((PLANE,)),
            pltpu.SemaphoreType.DMA((PLANE,)),
            pltpu.SemaphoreType.REGULAR,
        ],
        compiler_params=pltpu.CompilerParams(collective_id=0),
    )(x, route_idx, expert_W)
